# baseline (speedup 1.0000x reference)
import functools

import jax
import jax.numpy as jnp
from jax import lax
from jax.experimental import pallas as pl
from jax.experimental import layout as jlayout
from jax.experimental.pallas import tpu as pltpu
from jax.experimental.pallas import tpu_sc as plsc

NUM_CORES = 2
NUM_SUBCORES = 16
LANES = 16
NW = NUM_CORES * NUM_SUBCORES

BATCH = 16384
CHUNK = 128
BPW = BATCH // NW
ROWS_PW = BPW // CHUNK

def _flat128_layout():
    return jlayout.Layout(major_to_minor=(0,), tiling=((128,),))


@functools.partial(
    pl.kernel,
    mesh=plsc.VectorSubcoreMesh(core_axis_name="c", subcore_axis_name="s"),
    out_type=jax.ShapeDtypeStruct((BATCH,), jnp.float32),
    scratch_types=[
        pltpu.VMEM((BPW,), jnp.int32),
        pltpu.VMEM((BPW,), jnp.int32),
        pltpu.VMEM((ROWS_PW, CHUNK), jnp.float32),
        pltpu.VMEM((ROWS_PW, CHUNK), jnp.float32),
        pltpu.VMEM((BPW,), jnp.float32),
        pltpu.VMEM((LANES,), jnp.float32),
        pltpu.SemaphoreType.DMA,
    ],
    compiler_params=pltpu.CompilerParams(use_tc_tiling_on_sc=False),
)
def _nbm_kernel(user_hbm, movie_hbm, ubias_hbm, mbias_hbm, gb_hbm, out_hbm,
                uidx, midx, uval, mval, outv, gbv, sem):
    wid = lax.axis_index("s") * NUM_CORES + lax.axis_index("c")
    base = wid * BPW
    pltpu.sync_copy(user_hbm.at[pl.ds(base, BPW)], uidx)
    pltpu.sync_copy(movie_hbm.at[pl.ds(base, BPW)], midx)
    pltpu.sync_copy(gb_hbm, gbv)
    copies = []
    for j in range(ROWS_PW):
        isl = pl.ds(j * CHUNK, CHUNK)
        copies.append(pltpu.async_copy(ubias_hbm.at[uidx.at[isl]], uval.at[j], sem))
        copies.append(pltpu.async_copy(mbias_hbm.at[midx.at[isl]], mval.at[j], sem))
    for c in copies:
        c.wait()
    g = gbv[...]
    for j in range(ROWS_PW):
        for i in range(CHUNK // LANES):
            sl = pl.ds(i * LANES, LANES)
            outv[pl.ds(j * CHUNK + i * LANES, LANES)] = (
                uval[j, sl] + mval[j, sl] + g)
    pltpu.sync_copy(outv, out_hbm.at[pl.ds(base, BPW)])


def kernel(user, movie, user_biases, movie_biases, global_bias):
    lay = _flat128_layout()
    ub = jlayout.with_layout_constraint(user_biases.reshape(-1), lay)
    mb = jlayout.with_layout_constraint(movie_biases.reshape(-1), lay)
    gb = jnp.broadcast_to(global_bias.reshape(1), (LANES,))
    return _nbm_kernel(user, movie, ub, mb, gb)


# speedup vs baseline: 1.0020x; 1.0020x over previous
import functools

import jax
import jax.numpy as jnp
from jax import lax
from jax.experimental import pallas as pl
from jax.experimental import layout as jlayout
from jax.experimental.pallas import tpu as pltpu
from jax.experimental.pallas import tpu_sc as plsc

NUM_CORES = 2
NUM_SUBCORES = 16
LANES = 16
NW = NUM_CORES * NUM_SUBCORES

BATCH = 16384
CHUNK = 128
BPW = BATCH // NW
ROWS_PW = BPW // CHUNK

def _packed2d_layout():
    return jlayout.Layout(major_to_minor=(1, 0), tiling=((8, 1),))


@functools.partial(
    pl.kernel,
    mesh=plsc.VectorSubcoreMesh(core_axis_name="c", subcore_axis_name="s"),
    out_type=jax.ShapeDtypeStruct((BATCH,), jnp.float32),
    scratch_types=[
        pltpu.VMEM((BPW,), jnp.int32),
        pltpu.VMEM((BPW,), jnp.int32),
        pltpu.VMEM((ROWS_PW, CHUNK), jnp.float32),
        pltpu.VMEM((ROWS_PW, CHUNK), jnp.float32),
        pltpu.VMEM((BPW,), jnp.float32),
        pltpu.VMEM((LANES,), jnp.float32),
        pltpu.SemaphoreType.DMA,
    ],
    compiler_params=pltpu.CompilerParams(use_tc_tiling_on_sc=False),
)
def _nbm_kernel(user_hbm, movie_hbm, ubias_hbm, mbias_hbm, gb_hbm, out_hbm,
                uidx, midx, uval, mval, outv, gbv, sem):
    wid = lax.axis_index("s") * NUM_CORES + lax.axis_index("c")
    base = wid * BPW
    pltpu.sync_copy(user_hbm.at[pl.ds(base, BPW)], uidx)
    pltpu.sync_copy(movie_hbm.at[pl.ds(base, BPW)], midx)
    pltpu.sync_copy(gb_hbm, gbv)
    copies = []
    for j in range(ROWS_PW):
        isl = pl.ds(j * CHUNK, CHUNK)
        copies.append(pltpu.async_copy(ubias_hbm.at[uidx.at[isl]], uval.at[j], sem))
        copies.append(pltpu.async_copy(mbias_hbm.at[midx.at[isl]], mval.at[j], sem))
    for c in copies:
        c.wait()
    g = gbv[...]
    for j in range(ROWS_PW):
        for i in range(CHUNK // LANES):
            sl = pl.ds(i * LANES, LANES)
            outv[pl.ds(j * CHUNK + i * LANES, LANES)] = (
                uval[j, sl] + mval[j, sl] + g)
    pltpu.sync_copy(outv, out_hbm.at[pl.ds(base, BPW)])


def kernel(user, movie, user_biases, movie_biases, global_bias):
    lay = _packed2d_layout()
    ub = jlayout.with_layout_constraint(user_biases, lay).reshape(-1)
    mb = jlayout.with_layout_constraint(movie_biases, lay).reshape(-1)
    gb = jnp.broadcast_to(global_bias.reshape(1), (LANES,))
    return _nbm_kernel(user, movie, ub, mb, gb)
